# Initial kernel scaffold; baseline (speedup 1.0000x reference)
import functools

import jax
import jax.numpy as jnp
from jax import lax
from jax.experimental import pallas as pl
from jax.experimental.pallas import tpu as pltpu
from jax.experimental.pallas import tpu_sc as plsc

N_NODES = 10000
N_EDGES = 320000
F = 64
EPS_BN = 1e-5

NC, NS = 2, 16
NW = NC * NS
B_EDGE = 80
EPT = N_EDGES // NW
G_TILE = EPT // B_EDGE
ROWS_PER_TILE = N_NODES // NS
ZCHUNK = 125


def _sc_agg_body(hw, src2d, dst2d, out, agg, src_blk, dst_blk, rows, zbuf, sem):
    c = lax.axis_index("c")
    s = lax.axis_index("s")
    wid = c * NS + s

    pltpu.sync_copy(src2d.at[pl.ds(wid * G_TILE, G_TILE)], src_blk)
    pltpu.sync_copy(dst2d.at[pl.ds(wid * G_TILE, G_TILE)], dst_blk)

    zero = jnp.zeros((16,), jnp.float32)

    def zrow(r, carry):
        for c4 in range(F // 16):
            zbuf[r, pl.ds(c4 * 16, 16)] = zero
        return carry

    lax.fori_loop(0, ZCHUNK, zrow, 0)
    for i in range(ROWS_PER_TILE // ZCHUNK):
        pltpu.sync_copy(
            zbuf, agg.at[pl.ds(s * ROWS_PER_TILE + i * ZCHUNK, ZCHUNK)])
    plsc.subcore_barrier()

    def edge_group(g, carry):
        pltpu.async_copy(hw.at[src_blk.at[g]], rows, sem).wait()
        pltpu.sync_copy(rows, agg.at[dst_blk.at[g]], add=True)
        return carry

    lax.fori_loop(0, G_TILE, edge_group, 0)
    plsc.subcore_barrier()

    for i in range(ROWS_PER_TILE // ZCHUNK):
        sl = pl.ds(s * ROWS_PER_TILE + i * ZCHUNK, ZCHUNK)
        pltpu.sync_copy(agg.at[sl], zbuf)
        pltpu.sync_copy(zbuf, out.at[c].at[sl])


_sc_agg = functools.partial(
    pl.kernel,
    out_type=jax.ShapeDtypeStruct((NC, N_NODES, F), jnp.float32),
    mesh=plsc.VectorSubcoreMesh(
        core_axis_name="c", subcore_axis_name="s",
        num_cores=NC, num_subcores=NS),
    scratch_types=[
        pltpu.VMEM_SHARED((N_NODES, F), jnp.float32),
        pltpu.VMEM((G_TILE, B_EDGE), jnp.int32),
        pltpu.VMEM((G_TILE, B_EDGE), jnp.int32),
        pltpu.VMEM((B_EDGE, F), jnp.float32),
        pltpu.VMEM((ZCHUNK, F), jnp.float32),
        pltpu.SemaphoreType.DMA,
    ],
)(_sc_agg_body)


def _k1_body(x, w, b, wr, br, hw, res):
    xv = x[...]
    hw[...] = jnp.dot(xv, w[...], preferred_element_type=jnp.float32) + b[...]
    res[...] = jnp.maximum(
        jnp.dot(xv, wr[...], preferred_element_type=jnp.float32) + br[...], 0.0)


def _bn(t, g, be):
    mean = jnp.mean(t, axis=0, keepdims=True)
    var = jnp.mean(t * t, axis=0, keepdims=True) - mean * mean
    return (t - mean) * lax.rsqrt(var + EPS_BN) * g + be


def _k2_body(agg, res, g, be, w, b, wr, br, hw, res2):
    t = jnp.maximum(agg[0] + agg[1], 0.0) + res[...]
    h = _bn(t, g[...], be[...])
    hw[...] = jnp.dot(h, w[...], preferred_element_type=jnp.float32) + b[...]
    res2[...] = jnp.maximum(
        jnp.dot(h, wr[...], preferred_element_type=jnp.float32) + br[...], 0.0)


def _k3_body(agg, res, g, be, wd, bd, out):
    t = jnp.maximum(agg[0] + agg[1], 0.0) + res[...]
    h = _bn(t, g[...], be[...])
    logits = jnp.dot(h, wd[...], preferred_element_type=jnp.float32) + bd[...]
    m = jnp.max(logits, axis=1, keepdims=True)
    e = jnp.exp(logits - m)
    out[...] = e / jnp.sum(e, axis=1, keepdims=True)


def _tc_call(body, out_shapes):
    return pl.pallas_call(
        body,
        out_shape=[jax.ShapeDtypeStruct(s, jnp.float32) for s in out_shapes])


def kernel(in_feat, edge_index, W0, b0, Wr0, br0, g0, be0,
           W1, b1, Wr1, br1, g1, be1, Wd, bd):
    src2d = edge_index[0].reshape(NW * G_TILE, B_EDGE)
    dst2d = edge_index[1].reshape(NW * G_TILE, B_EDGE)

    hw0, res0 = _tc_call(_k1_body, [(N_NODES, F), (N_NODES, F)])(
        in_feat, W0, b0.reshape(1, F), Wr0, br0.reshape(1, F))
    agg0 = _sc_agg(hw0, src2d, dst2d)

    hw1, res1 = _tc_call(_k2_body, [(N_NODES, F), (N_NODES, F)])(
        agg0, res0, g0.reshape(1, F), be0.reshape(1, F),
        W1, b1.reshape(1, F), Wr1, br1.reshape(1, F))
    agg1 = _sc_agg(hw1, src2d, dst2d)

    out, = _tc_call(_k3_body, [(N_NODES, 2)])(
        agg1, res1, g1.reshape(1, F), be1.reshape(1, F),
        Wd, bd.reshape(1, 2))
    return out


# speedup vs baseline: 6.4136x; 6.4136x over previous
import functools

import jax
import jax.numpy as jnp
from jax import lax
from jax.experimental import pallas as pl
from jax.experimental.pallas import tpu as pltpu
from jax.experimental.pallas import tpu_sc as plsc

N_NODES = 10000
N_EDGES = 320000
F = 64
FP = 128
EPS_BN = 1e-5

NC, NS = 2, 16
NW = NC * NS
B_EDGE = 80
EPT = N_EDGES // NW
G_TILE = EPT // B_EDGE
NCH = 5
G_CHUNK = G_TILE // NCH
AGG_ROWS = 10240
ROWS_PER_TILE = AGG_ROWS // NS
ZCHUNK = 64


def _sc_agg_body(hw, src4d, dst4d, out, agg, src_blk, dst_blk, rows, zbuf, sem):
    c = lax.axis_index("c")
    s = lax.axis_index("s")
    wid = c * NS + s

    zero = jnp.zeros((16,), jnp.float32)

    def zrow(r, carry):
        for c16 in range(FP // 16):
            zbuf[r, pl.ds(c16 * 16, 16)] = zero
        return carry

    lax.fori_loop(0, ZCHUNK, zrow, 0)
    for i in range(ROWS_PER_TILE // ZCHUNK):
        pltpu.sync_copy(
            zbuf, agg.at[pl.ds(s * ROWS_PER_TILE + i * ZCHUNK, ZCHUNK)])
    plsc.subcore_barrier()

    def edge_group(g, carry):
        pltpu.async_copy(hw.at[src_blk.at[g]], rows, sem).wait()
        pltpu.sync_copy(rows, agg.at[dst_blk.at[g]], add=True)
        return carry

    for ch in range(NCH):
        pltpu.sync_copy(src4d.at[wid, ch], src_blk)
        pltpu.sync_copy(dst4d.at[wid, ch], dst_blk)
        lax.fori_loop(0, G_CHUNK, edge_group, 0)
    plsc.subcore_barrier()

    for i in range(ROWS_PER_TILE // ZCHUNK):
        sl = pl.ds(s * ROWS_PER_TILE + i * ZCHUNK, ZCHUNK)
        pltpu.sync_copy(agg.at[sl], zbuf)
        pltpu.sync_copy(zbuf, out.at[c].at[sl])


@functools.cache
def _sc_agg_kernel():
    return pl.kernel(
        _sc_agg_body,
        out_type=jax.ShapeDtypeStruct((NC, AGG_ROWS, FP), jnp.float32),
        mesh=plsc.VectorSubcoreMesh(
            core_axis_name="c", subcore_axis_name="s",
            num_cores=NC, num_subcores=NS),
        scratch_types=[
            pltpu.VMEM_SHARED((AGG_ROWS, FP), jnp.float32),
            pltpu.VMEM((G_CHUNK, B_EDGE), jnp.int32),
            pltpu.VMEM((G_CHUNK, B_EDGE), jnp.int32),
            pltpu.VMEM((B_EDGE, FP), jnp.float32),
            pltpu.VMEM((ZCHUNK, FP), jnp.float32),
            pltpu.SemaphoreType.DMA,
        ],
    )


def _sc_agg(hw, src2d, dst2d):
    return _sc_agg_kernel()(hw, src2d, dst2d)


def _k1_body(x, w, b, wr, br, hw, res):
    xv = x[...]
    hw[...] = jnp.dot(xv, w[...], preferred_element_type=jnp.float32) + b[...]
    res[...] = jnp.maximum(
        jnp.dot(xv, wr[...], preferred_element_type=jnp.float32) + br[...], 0.0)


def _bn(t, g, be):
    mean = jnp.mean(t, axis=0, keepdims=True)
    var = jnp.mean(t * t, axis=0, keepdims=True) - mean * mean
    return (t - mean) * lax.rsqrt(var + EPS_BN) * g + be


def _k2_body(agg, res, g, be, w, b, wr, br, hw, res2):
    t = jnp.maximum(agg[0] + agg[1], 0.0) + res[...]
    h = _bn(t, g[...], be[...])
    hw[...] = jnp.dot(h, w[...], preferred_element_type=jnp.float32) + b[...]
    res2[...] = jnp.maximum(
        jnp.dot(h, wr[...], preferred_element_type=jnp.float32) + br[...], 0.0)


def _k3_body(agg, res, g, be, wd, bd, out):
    t = jnp.maximum(agg[0] + agg[1], 0.0) + res[...]
    h = _bn(t, g[...], be[...])
    logits = jnp.dot(h, wd[...], preferred_element_type=jnp.float32) + bd[...]
    m = jnp.max(logits, axis=1, keepdims=True)
    e = jnp.exp(logits - m)
    out[...] = e / jnp.sum(e, axis=1, keepdims=True)


def _tc_call(body, out_shapes):
    return pl.pallas_call(
        body,
        out_shape=[jax.ShapeDtypeStruct(s, jnp.float32) for s in out_shapes])


def _pad_cols(a, width=FP):
    return jnp.pad(a, ((0, 0), (0, width - a.shape[1])))


def kernel(in_feat, edge_index, W0, b0, Wr0, br0, g0, be0,
           W1, b1, Wr1, br1, g1, be1, Wd, bd):
    src2d = edge_index[0].reshape(NW, NCH, G_CHUNK, B_EDGE)
    dst2d = edge_index[1].reshape(NW, NCH, G_CHUNK, B_EDGE)
    b0p = _pad_cols(b0.reshape(1, F))
    b1p = _pad_cols(b1.reshape(1, F))

    hw0, res0 = _tc_call(_k1_body, [(N_NODES, FP), (N_NODES, F)])(
        in_feat, _pad_cols(W0), b0p, Wr0, br0.reshape(1, F))
    agg0 = _sc_agg(hw0, src2d, dst2d)[:, :N_NODES, :F]

    hw1, res1 = _tc_call(_k2_body, [(N_NODES, FP), (N_NODES, F)])(
        agg0, res0, g0.reshape(1, F), be0.reshape(1, F),
        _pad_cols(W1), b1p, Wr1, br1.reshape(1, F))
    agg1 = _sc_agg(hw1, src2d, dst2d)[:, :N_NODES, :F]

    out, = _tc_call(_k3_body, [(N_NODES, 2)])(
        agg1, res1, g1.reshape(1, F), be1.reshape(1, F),
        Wd, bd.reshape(1, 2))
    return out


# speedup vs baseline: 7.9813x; 1.2444x over previous
import functools

import jax
import jax.numpy as jnp
from jax import lax
from jax.experimental import pallas as pl
from jax.experimental.pallas import tpu as pltpu
from jax.experimental.pallas import tpu_sc as plsc

N_NODES = 10000
N_EDGES = 320000
F = 64
FP = 64
EPS_BN = 1e-5

NC, NS = 2, 16
NW = NC * NS
B_EDGE = 80
EPT = N_EDGES // NW
G_TILE = EPT // B_EDGE
NCH = 5
G_CHUNK = G_TILE // NCH
AGG_ROWS = 10240
ROWS_PER_TILE = AGG_ROWS // NS
ZCHUNK = 64


def _sc_agg_body(hw, src4d, dst4d, out, agg, src_blk, dst_blk, rows, zbuf, sem):
    c = lax.axis_index("c")
    s = lax.axis_index("s")
    wid = c * NS + s

    zero = jnp.zeros((16,), jnp.float32)

    def zrow(r, carry):
        for c16 in range(FP // 16):
            zbuf[r, pl.ds(c16 * 16, 16)] = zero
        return carry

    lax.fori_loop(0, ZCHUNK, zrow, 0)
    for i in range(ROWS_PER_TILE // ZCHUNK):
        pltpu.sync_copy(
            zbuf, agg.at[pl.ds(s * ROWS_PER_TILE + i * ZCHUNK, ZCHUNK)])
    plsc.subcore_barrier()

    def edge_group(g, carry):
        pltpu.async_copy(hw.at[src_blk.at[g]], rows, sem).wait()
        pltpu.sync_copy(rows, agg.at[dst_blk.at[g]], add=True)
        return carry

    for ch in range(NCH):
        pltpu.sync_copy(src4d.at[wid, ch], src_blk)
        pltpu.sync_copy(dst4d.at[wid, ch], dst_blk)
        lax.fori_loop(0, G_CHUNK, edge_group, 0)
    plsc.subcore_barrier()

    for i in range(ROWS_PER_TILE // ZCHUNK):
        sl = pl.ds(s * ROWS_PER_TILE + i * ZCHUNK, ZCHUNK)
        pltpu.sync_copy(agg.at[sl], zbuf)
        pltpu.sync_copy(zbuf, out.at[c].at[sl])


@functools.cache
def _sc_agg_kernel():
    return pl.kernel(
        _sc_agg_body,
        out_type=jax.ShapeDtypeStruct((NC, AGG_ROWS, FP), jnp.float32),
        mesh=plsc.VectorSubcoreMesh(
            core_axis_name="c", subcore_axis_name="s",
            num_cores=NC, num_subcores=NS),
        compiler_params=pltpu.CompilerParams(use_tc_tiling_on_sc=False),
        scratch_types=[
            pltpu.VMEM_SHARED((AGG_ROWS, FP), jnp.float32),
            pltpu.VMEM((G_CHUNK, B_EDGE), jnp.int32),
            pltpu.VMEM((G_CHUNK, B_EDGE), jnp.int32),
            pltpu.VMEM((B_EDGE, FP), jnp.float32),
            pltpu.VMEM((ZCHUNK, FP), jnp.float32),
            pltpu.SemaphoreType.DMA,
        ],
    )


def _sc_agg(hw, src2d, dst2d):
    return _sc_agg_kernel()(hw, src2d, dst2d)


def _k1_body(x, w, b, wr, br, hw, res):
    xv = x[...]
    hw[...] = jnp.dot(xv, w[...], preferred_element_type=jnp.float32) + b[...]
    res[...] = jnp.maximum(
        jnp.dot(xv, wr[...], preferred_element_type=jnp.float32) + br[...], 0.0)


def _bn(t, g, be):
    mean = jnp.mean(t, axis=0, keepdims=True)
    var = jnp.mean(t * t, axis=0, keepdims=True) - mean * mean
    return (t - mean) * lax.rsqrt(var + EPS_BN) * g + be


def _k2_body(agg, res, g, be, w, b, wr, br, hw, res2):
    t = jnp.maximum(agg[0] + agg[1], 0.0) + res[...]
    h = _bn(t, g[...], be[...])
    hw[...] = jnp.dot(h, w[...], preferred_element_type=jnp.float32) + b[...]
    res2[...] = jnp.maximum(
        jnp.dot(h, wr[...], preferred_element_type=jnp.float32) + br[...], 0.0)


def _k3_body(agg, res, g, be, wd, bd, out):
    t = jnp.maximum(agg[0] + agg[1], 0.0) + res[...]
    h = _bn(t, g[...], be[...])
    logits = jnp.dot(h, wd[...], preferred_element_type=jnp.float32) + bd[...]
    m = jnp.max(logits, axis=1, keepdims=True)
    e = jnp.exp(logits - m)
    out[...] = e / jnp.sum(e, axis=1, keepdims=True)


def _tc_call(body, out_shapes):
    return pl.pallas_call(
        body,
        out_shape=[jax.ShapeDtypeStruct(s, jnp.float32) for s in out_shapes])


def _pad_cols(a, width=FP):
    return jnp.pad(a, ((0, 0), (0, width - a.shape[1])))


def kernel(in_feat, edge_index, W0, b0, Wr0, br0, g0, be0,
           W1, b1, Wr1, br1, g1, be1, Wd, bd):
    src2d = edge_index[0].reshape(NW, NCH, G_CHUNK, B_EDGE)
    dst2d = edge_index[1].reshape(NW, NCH, G_CHUNK, B_EDGE)
    b0p = _pad_cols(b0.reshape(1, F))
    b1p = _pad_cols(b1.reshape(1, F))

    hw0, res0 = _tc_call(_k1_body, [(N_NODES, FP), (N_NODES, F)])(
        in_feat, _pad_cols(W0), b0p, Wr0, br0.reshape(1, F))
    agg0 = _sc_agg(hw0, src2d, dst2d)[:, :N_NODES, :F]

    hw1, res1 = _tc_call(_k2_body, [(N_NODES, FP), (N_NODES, F)])(
        agg0, res0, g0.reshape(1, F), be0.reshape(1, F),
        _pad_cols(W1), b1p, Wr1, br1.reshape(1, F))
    agg1 = _sc_agg(hw1, src2d, dst2d)[:, :N_NODES, :F]

    out, = _tc_call(_k3_body, [(N_NODES, 2)])(
        agg1, res1, g1.reshape(1, F), be1.reshape(1, F),
        Wd, bd.reshape(1, 2))
    return out


# speedup vs baseline: 15.3587x; 1.9243x over previous
import functools

import jax
import jax.numpy as jnp
from jax import lax
from jax.experimental import pallas as pl
from jax.experimental.pallas import tpu as pltpu
from jax.experimental.pallas import tpu_sc as plsc

N_NODES = 10000
N_EDGES = 320000
F = 64
FP = 64
EPS_BN = 1e-5

NC, NS = 2, 16
NW = NC * NS
B_EDGE = 80
EPT = N_EDGES // NW
G_TILE = EPT // B_EDGE
NBUF = 5
AGG_ROWS = 10240
ROWS_PER_TILE = AGG_ROWS // NS
ZCHUNK = 64


def _sc_agg_body(hw, src3d, dst3d, out, agg, src_blk, dst_blk, rows, zbuf, sem):
    c = lax.axis_index("c")
    s = lax.axis_index("s")
    wid = c * NS + s

    pltpu.sync_copy(src3d.at[wid], src_blk)
    pltpu.sync_copy(dst3d.at[wid], dst_blk)

    zero = jnp.zeros((16,), jnp.float32)

    def zrow(r, carry):
        for c16 in range(FP // 16):
            zbuf[r, pl.ds(c16 * 16, 16)] = zero
        return carry

    lax.fori_loop(0, ZCHUNK, zrow, 0)
    for i in range(ROWS_PER_TILE // ZCHUNK):
        pltpu.sync_copy(
            zbuf, agg.at[pl.ds(s * ROWS_PER_TILE + i * ZCHUNK, ZCHUNK)])
    plsc.subcore_barrier()

    def _wait_gather(b):
        pltpu.make_async_copy(
            hw.at[pl.ds(0, B_EDGE)], rows.at[b], sem.at[b]).wait()

    for b in range(NBUF):
        pltpu.async_copy(hw.at[src_blk.at[b]], rows.at[b], sem.at[b])

    def step(j, carry):
        for b in range(NBUF):
            g = j * NBUF + b
            _wait_gather(b)
            pltpu.sync_copy(rows.at[b], agg.at[dst_blk.at[g]], add=True)
            pltpu.async_copy(hw.at[src_blk.at[g + NBUF]], rows.at[b],
                             sem.at[b])
        return carry

    lax.fori_loop(0, G_TILE // NBUF - 1, step, 0)
    for b in range(NBUF):
        g = G_TILE - NBUF + b
        _wait_gather(b)
        pltpu.sync_copy(rows.at[b], agg.at[dst_blk.at[g]], add=True)
    plsc.subcore_barrier()

    for i in range(ROWS_PER_TILE // ZCHUNK):
        sl = pl.ds(s * ROWS_PER_TILE + i * ZCHUNK, ZCHUNK)
        pltpu.sync_copy(agg.at[sl], zbuf)
        pltpu.sync_copy(zbuf, out.at[c].at[sl])


@functools.cache
def _sc_agg_kernel():
    return pl.kernel(
        _sc_agg_body,
        out_type=jax.ShapeDtypeStruct((NC, AGG_ROWS, FP), jnp.float32),
        mesh=plsc.VectorSubcoreMesh(
            core_axis_name="c", subcore_axis_name="s",
            num_cores=NC, num_subcores=NS),
        compiler_params=pltpu.CompilerParams(use_tc_tiling_on_sc=False),
        scratch_types=[
            pltpu.VMEM_SHARED((AGG_ROWS, FP), jnp.float32),
            pltpu.VMEM((G_TILE, B_EDGE), jnp.int32),
            pltpu.VMEM((G_TILE, B_EDGE), jnp.int32),
            pltpu.VMEM((NBUF, B_EDGE, FP), jnp.float32),
            pltpu.VMEM((ZCHUNK, FP), jnp.float32),
            pltpu.SemaphoreType.DMA((NBUF,)),
        ],
    )


def _sc_agg(hw, src2d, dst2d):
    return _sc_agg_kernel()(hw, src2d, dst2d)


def _k1_body(x, w, b, wr, br, hw, res):
    xv = x[...]
    hw[...] = jnp.dot(xv, w[...], preferred_element_type=jnp.float32) + b[...]
    res[...] = jnp.maximum(
        jnp.dot(xv, wr[...], preferred_element_type=jnp.float32) + br[...], 0.0)


def _bn(t, g, be):
    mean = jnp.mean(t, axis=0, keepdims=True)
    var = jnp.mean(t * t, axis=0, keepdims=True) - mean * mean
    return (t - mean) * lax.rsqrt(var + EPS_BN) * g + be


def _k2_body(agg, res, g, be, w, b, wr, br, hw, res2):
    t = jnp.maximum(agg[0] + agg[1], 0.0) + res[...]
    h = _bn(t, g[...], be[...])
    hw[...] = jnp.dot(h, w[...], preferred_element_type=jnp.float32) + b[...]
    res2[...] = jnp.maximum(
        jnp.dot(h, wr[...], preferred_element_type=jnp.float32) + br[...], 0.0)


def _k3_body(agg, res, g, be, wd, bd, out):
    t = jnp.maximum(agg[0] + agg[1], 0.0) + res[...]
    h = _bn(t, g[...], be[...])
    logits = jnp.dot(h, wd[...], preferred_element_type=jnp.float32) + bd[...]
    m = jnp.max(logits, axis=1, keepdims=True)
    e = jnp.exp(logits - m)
    out[...] = e / jnp.sum(e, axis=1, keepdims=True)


def _tc_call(body, out_shapes):
    return pl.pallas_call(
        body,
        out_shape=[jax.ShapeDtypeStruct(s, jnp.float32) for s in out_shapes])


def _pad_cols(a, width=FP):
    return jnp.pad(a, ((0, 0), (0, width - a.shape[1])))


def kernel(in_feat, edge_index, W0, b0, Wr0, br0, g0, be0,
           W1, b1, Wr1, br1, g1, be1, Wd, bd):
    src2d = edge_index[0].reshape(NW, G_TILE, B_EDGE)
    dst2d = edge_index[1].reshape(NW, G_TILE, B_EDGE)
    b0p = _pad_cols(b0.reshape(1, F))
    b1p = _pad_cols(b1.reshape(1, F))

    hw0, res0 = _tc_call(_k1_body, [(N_NODES, FP), (N_NODES, F)])(
        in_feat, _pad_cols(W0), b0p, Wr0, br0.reshape(1, F))
    agg0 = _sc_agg(hw0, src2d, dst2d)[:, :N_NODES, :F]

    hw1, res1 = _tc_call(_k2_body, [(N_NODES, FP), (N_NODES, F)])(
        agg0, res0, g0.reshape(1, F), be0.reshape(1, F),
        _pad_cols(W1), b1p, Wr1, br1.reshape(1, F))
    agg1 = _sc_agg(hw1, src2d, dst2d)[:, :N_NODES, :F]

    out, = _tc_call(_k3_body, [(N_NODES, 2)])(
        agg1, res1, g1.reshape(1, F), be1.reshape(1, F),
        Wd, bd.reshape(1, 2))
    return out


# speedup vs baseline: 17.3269x; 1.1281x over previous
import functools

import jax
import jax.numpy as jnp
from jax import lax
from jax.experimental import pallas as pl
from jax.experimental.pallas import tpu as pltpu
from jax.experimental.pallas import tpu_sc as plsc

N_NODES = 10000
N_EDGES = 320000
F = 64
FP = 64
EPS_BN = 1e-5

NC, NS = 2, 16
NW = NC * NS
B_EDGE = 80
EPT = N_EDGES // NW
G_TILE = EPT // B_EDGE
NBUF = 5
AGG_ROWS = 10240
ROWS_PER_TILE = AGG_ROWS // NS
ZCHUNK = 64


def _sc_agg_body(hw, ei4, out, agg, src_blk, dst_blk, rows, zbuf, sem):
    c = lax.axis_index("c")
    s = lax.axis_index("s")
    wid = c * NS + s

    pltpu.sync_copy(ei4.at[0, wid], src_blk)
    pltpu.sync_copy(ei4.at[1, wid], dst_blk)

    zero = jnp.zeros((16,), jnp.float32)

    def zrow(r, carry):
        for c16 in range(FP // 16):
            zbuf[r, pl.ds(c16 * 16, 16)] = zero
        return carry

    lax.fori_loop(0, ZCHUNK, zrow, 0)
    for i in range(ROWS_PER_TILE // ZCHUNK):
        pltpu.sync_copy(
            zbuf, agg.at[pl.ds(s * ROWS_PER_TILE + i * ZCHUNK, ZCHUNK)])
    plsc.subcore_barrier()

    def _wait_gather(b):
        pltpu.make_async_copy(
            hw.at[pl.ds(0, B_EDGE)], rows.at[b], sem.at[b]).wait()

    for b in range(NBUF):
        pltpu.async_copy(hw.at[src_blk.at[b]], rows.at[b], sem.at[b])

    def step(j, carry):
        for b in range(NBUF):
            g = j * NBUF + b
            _wait_gather(b)
            pltpu.sync_copy(rows.at[b], agg.at[dst_blk.at[g]], add=True)
            pltpu.async_copy(hw.at[src_blk.at[g + NBUF]], rows.at[b],
                             sem.at[b])
        return carry

    lax.fori_loop(0, G_TILE // NBUF - 1, step, 0)
    for b in range(NBUF):
        g = G_TILE - NBUF + b
        _wait_gather(b)
        pltpu.sync_copy(rows.at[b], agg.at[dst_blk.at[g]], add=True)
    plsc.subcore_barrier()

    for i in range(ROWS_PER_TILE // ZCHUNK):
        sl = pl.ds(s * ROWS_PER_TILE + i * ZCHUNK, ZCHUNK)
        pltpu.sync_copy(agg.at[sl], zbuf)
        pltpu.sync_copy(zbuf, out.at[c].at[sl])


@functools.cache
def _sc_agg_kernel():
    return pl.kernel(
        _sc_agg_body,
        out_type=jax.ShapeDtypeStruct((NC, AGG_ROWS, FP), jnp.float32),
        mesh=plsc.VectorSubcoreMesh(
            core_axis_name="c", subcore_axis_name="s",
            num_cores=NC, num_subcores=NS),
        compiler_params=pltpu.CompilerParams(use_tc_tiling_on_sc=False),
        scratch_types=[
            pltpu.VMEM_SHARED((AGG_ROWS, FP), jnp.float32),
            pltpu.VMEM((G_TILE, B_EDGE), jnp.int32),
            pltpu.VMEM((G_TILE, B_EDGE), jnp.int32),
            pltpu.VMEM((NBUF, B_EDGE, FP), jnp.float32),
            pltpu.VMEM((ZCHUNK, FP), jnp.float32),
            pltpu.SemaphoreType.DMA((NBUF,)),
        ],
    )


def _sc_agg(hw, ei4):
    return _sc_agg_kernel()(hw, ei4)


def _k1_body(x, w, b, wr, br, hw, res):
    xv = x[...]
    hw[...] = jnp.dot(xv, w[...], preferred_element_type=jnp.float32) + b[...]
    res[...] = jnp.maximum(
        jnp.dot(xv, wr[...], preferred_element_type=jnp.float32) + br[...], 0.0)


def _bn(t, g, be):
    mean = jnp.mean(t, axis=0, keepdims=True)
    var = jnp.mean(t * t, axis=0, keepdims=True) - mean * mean
    return (t - mean) * lax.rsqrt(var + EPS_BN) * g + be


def _k2_body(agg, res, g, be, w, b, wr, br, hw, res2):
    t = jnp.maximum(agg[0, :N_NODES] + agg[1, :N_NODES], 0.0) + res[...]
    h = _bn(t, g[...], be[...])
    hw[...] = jnp.dot(h, w[...], preferred_element_type=jnp.float32) + b[...]
    res2[...] = jnp.maximum(
        jnp.dot(h, wr[...], preferred_element_type=jnp.float32) + br[...], 0.0)


def _k3_body(agg, res, g, be, wd, bd, out):
    t = jnp.maximum(agg[0, :N_NODES] + agg[1, :N_NODES], 0.0) + res[...]
    h = _bn(t, g[...], be[...])
    logits = jnp.dot(h, wd[...], preferred_element_type=jnp.float32) + bd[...]
    m = jnp.max(logits, axis=1, keepdims=True)
    e = jnp.exp(logits - m)
    out[...] = e / jnp.sum(e, axis=1, keepdims=True)


def _tc_call(body, out_shapes):
    return pl.pallas_call(
        body,
        out_shape=[jax.ShapeDtypeStruct(s, jnp.float32) for s in out_shapes])


def _pad_cols(a, width=FP):
    return jnp.pad(a, ((0, 0), (0, width - a.shape[1])))


def kernel(in_feat, edge_index, W0, b0, Wr0, br0, g0, be0,
           W1, b1, Wr1, br1, g1, be1, Wd, bd):
    ei4 = edge_index.reshape(2, NW, G_TILE, B_EDGE)
    b0p = _pad_cols(b0.reshape(1, F))
    b1p = _pad_cols(b1.reshape(1, F))

    hw0, res0 = _tc_call(_k1_body, [(N_NODES, FP), (N_NODES, F)])(
        in_feat, _pad_cols(W0), b0p, Wr0, br0.reshape(1, F))
    agg0 = _sc_agg(hw0, ei4)

    hw1, res1 = _tc_call(_k2_body, [(N_NODES, FP), (N_NODES, F)])(
        agg0, res0, g0.reshape(1, F), be0.reshape(1, F),
        _pad_cols(W1), b1p, Wr1, br1.reshape(1, F))
    agg1 = _sc_agg(hw1, ei4)

    out, = _tc_call(_k3_body, [(N_NODES, 2)])(
        agg1, res1, g1.reshape(1, F), be1.reshape(1, F),
        Wd, bd.reshape(1, 2))
    return out


# speedup vs baseline: 17.4648x; 1.0080x over previous
import functools

import jax
import jax.numpy as jnp
from jax import lax
from jax.experimental import pallas as pl
from jax.experimental.pallas import tpu as pltpu
from jax.experimental.pallas import tpu_sc as plsc

N_NODES = 10000
N_EDGES = 320000
F = 64
FP = 64
EPS_BN = 1e-5

NC, NS = 2, 16
NW = NC * NS
B_EDGE = 80
EPT = N_EDGES // NW
G_TILE = EPT // B_EDGE
NBUF = 5
AGG_ROWS = 10240
ROWS_PER_TILE = AGG_ROWS // NS
ZCHUNK = 64


def _sc_agg_body(hw, ei4, out, agg, src_blk, dst_blk, rows, zbuf, sem):
    c = lax.axis_index("c")
    s = lax.axis_index("s")
    wid = c * NS + s

    pltpu.sync_copy(ei4.at[0, wid], src_blk)
    pltpu.sync_copy(ei4.at[1, wid], dst_blk)

    zero = jnp.zeros((16,), jnp.float32)

    def zrow(r, carry):
        for c16 in range(FP // 16):
            zbuf[r, pl.ds(c16 * 16, 16)] = zero
        return carry

    lax.fori_loop(0, ZCHUNK, zrow, 0)
    for i in range(ROWS_PER_TILE // ZCHUNK):
        pltpu.sync_copy(
            zbuf, agg.at[pl.ds(s * ROWS_PER_TILE + i * ZCHUNK, ZCHUNK)])
    plsc.subcore_barrier()

    def _wait_gather(b):
        pltpu.make_async_copy(
            hw.at[pl.ds(0, B_EDGE)], rows.at[b], sem.at[b]).wait()

    for b in range(NBUF):
        pltpu.async_copy(hw.at[src_blk.at[b]], rows.at[b], sem.at[b])

    def step(j, carry):
        for b in range(NBUF):
            g = j * NBUF + b
            _wait_gather(b)
            pltpu.sync_copy(rows.at[b], agg.at[dst_blk.at[g]], add=True)
            pltpu.async_copy(hw.at[src_blk.at[g + NBUF]], rows.at[b],
                             sem.at[b])
        return carry

    lax.fori_loop(0, G_TILE // NBUF - 1, step, 0)
    for b in range(NBUF):
        g = G_TILE - NBUF + b
        _wait_gather(b)
        pltpu.sync_copy(rows.at[b], agg.at[dst_blk.at[g]], add=True)
    plsc.subcore_barrier()

    for i in range(ROWS_PER_TILE // ZCHUNK):
        sl = pl.ds(s * ROWS_PER_TILE + i * ZCHUNK, ZCHUNK)
        pltpu.sync_copy(agg.at[sl], zbuf)
        pltpu.sync_copy(zbuf, out.at[c].at[sl])


@functools.cache
def _sc_agg_kernel():
    return pl.kernel(
        _sc_agg_body,
        out_type=jax.ShapeDtypeStruct((NC, AGG_ROWS, FP), jnp.float32),
        mesh=plsc.VectorSubcoreMesh(
            core_axis_name="c", subcore_axis_name="s",
            num_cores=NC, num_subcores=NS),
        compiler_params=pltpu.CompilerParams(use_tc_tiling_on_sc=False),
        scratch_types=[
            pltpu.VMEM_SHARED((AGG_ROWS, FP), jnp.float32),
            pltpu.VMEM((G_TILE, B_EDGE), jnp.int32),
            pltpu.VMEM((G_TILE, B_EDGE), jnp.int32),
            pltpu.VMEM((NBUF, B_EDGE, FP), jnp.float32),
            pltpu.VMEM((ZCHUNK, FP), jnp.float32),
            pltpu.SemaphoreType.DMA((NBUF,)),
        ],
    )


def _sc_agg(hw, ei4):
    return _sc_agg_kernel()(hw, ei4)


def _mm_body(x, w, b, hw):
    hw[...] = jnp.dot(x[...], w[...], preferred_element_type=jnp.float32) + b[...]


def _mm_relu_body(x, w, b, res):
    res[...] = jnp.maximum(
        jnp.dot(x[...], w[...], preferred_element_type=jnp.float32) + b[...], 0.0)


def _bn(t, g, be):
    mean = jnp.mean(t, axis=0, keepdims=True)
    var = jnp.mean(t * t, axis=0, keepdims=True) - mean * mean
    return (t - mean) * lax.rsqrt(var + EPS_BN) * g + be


def _k2_body(agg, res, g, be, w, b, hw, hout):
    t = jnp.maximum(agg[0, :N_NODES] + agg[1, :N_NODES], 0.0) + res[...]
    h = _bn(t, g[...], be[...])
    hw[...] = jnp.dot(h, w[...], preferred_element_type=jnp.float32) + b[...]
    hout[...] = h


def _k3_body(agg, res, g, be, wd, bd, out):
    t = jnp.maximum(agg[0, :N_NODES] + agg[1, :N_NODES], 0.0) + res[...]
    h = _bn(t, g[...], be[...])
    logits = jnp.dot(h, wd[...], preferred_element_type=jnp.float32) + bd[...]
    m = jnp.max(logits, axis=1, keepdims=True)
    e = jnp.exp(logits - m)
    out[...] = e / jnp.sum(e, axis=1, keepdims=True)


def _tc_call(body, out_shapes):
    return pl.pallas_call(
        body,
        out_shape=[jax.ShapeDtypeStruct(s, jnp.float32) for s in out_shapes])


def _pad_cols(a, width=FP):
    return jnp.pad(a, ((0, 0), (0, width - a.shape[1])))


def kernel(in_feat, edge_index, W0, b0, Wr0, br0, g0, be0,
           W1, b1, Wr1, br1, g1, be1, Wd, bd):
    ei4 = edge_index.reshape(2, NW, G_TILE, B_EDGE)
    b0p = _pad_cols(b0.reshape(1, F))
    b1p = _pad_cols(b1.reshape(1, F))

    hw0, = _tc_call(_mm_body, [(N_NODES, FP)])(in_feat, _pad_cols(W0), b0p)
    agg0 = _sc_agg(hw0, ei4)
    res0, = _tc_call(_mm_relu_body, [(N_NODES, F)])(
        in_feat, Wr0, br0.reshape(1, F))

    hw1, h1 = _tc_call(_k2_body, [(N_NODES, FP), (N_NODES, F)])(
        agg0, res0, g0.reshape(1, F), be0.reshape(1, F), _pad_cols(W1), b1p)
    agg1 = _sc_agg(hw1, ei4)
    res1, = _tc_call(_mm_relu_body, [(N_NODES, F)])(
        h1, Wr1, br1.reshape(1, F))

    out, = _tc_call(_k3_body, [(N_NODES, 2)])(
        agg1, res1, g1.reshape(1, F), be1.reshape(1, F),
        Wd, bd.reshape(1, 2))
    return out


# speedup vs baseline: 18.8867x; 1.0814x over previous
import functools

import jax
import jax.numpy as jnp
from jax import lax
from jax.experimental import pallas as pl
from jax.experimental.pallas import tpu as pltpu
from jax.experimental.pallas import tpu_sc as plsc

N_NODES = 10000
N_EDGES = 320000
F = 64
FP = 64
EPS_BN = 1e-5

NC, NS = 2, 16
NW = NC * NS
B_EDGE = 80
EPT = N_EDGES // NW
G_TILE = EPT // B_EDGE
NBUF = 5
AGG_ROWS = 10240
ROWS_PER_TILE = AGG_ROWS // NS
ZCHUNK = 64
FD = 128


def _sc_agg_body(hw, ei4, out, agg, src_blk, dst_blk, rows, zbuf, dbuf, sem):
    c = lax.axis_index("c")
    s = lax.axis_index("s")
    wid = c * NS + s

    pltpu.sync_copy(ei4.at[0, wid], src_blk)
    pltpu.sync_copy(ei4.at[1, wid], dst_blk)

    zero = jnp.zeros((16,), jnp.float32)

    def zrow(r, carry):
        for c16 in range(FP // 16):
            zbuf[r, pl.ds(c16 * 16, 16)] = zero
        return carry

    lax.fori_loop(0, ZCHUNK, zrow, 0)

    def zrow_d(r, carry):
        for c16 in range(FD // 16):
            dbuf[r, pl.ds(c16 * 16, 16)] = zero
        return carry

    lax.fori_loop(0, ZCHUNK, zrow_d, 0)
    for i in range(ROWS_PER_TILE // ZCHUNK):
        pltpu.sync_copy(
            zbuf, agg.at[pl.ds(s * ROWS_PER_TILE + i * ZCHUNK, ZCHUNK)])
    plsc.subcore_barrier()

    def _wait_gather(b):
        pltpu.make_async_copy(
            hw.at[pl.ds(0, B_EDGE)], rows.at[b], sem.at[b]).wait()

    for b in range(NBUF):
        pltpu.async_copy(hw.at[src_blk.at[b]], rows.at[b], sem.at[b])

    def step(j, carry):
        for b in range(NBUF):
            g = j * NBUF + b
            _wait_gather(b)
            pltpu.sync_copy(rows.at[b], agg.at[dst_blk.at[g]], add=True)
            pltpu.async_copy(hw.at[src_blk.at[g + NBUF]], rows.at[b],
                             sem.at[b])
        return carry

    lax.fori_loop(0, G_TILE // NBUF - 1, step, 0)
    for b in range(NBUF):
        g = G_TILE - NBUF + b
        _wait_gather(b)
        pltpu.sync_copy(rows.at[b], agg.at[dst_blk.at[g]], add=True)
    plsc.subcore_barrier()

    for i in range(ROWS_PER_TILE // ZCHUNK):
        sl = pl.ds(s * ROWS_PER_TILE + i * ZCHUNK, ZCHUNK)
        pltpu.sync_copy(agg.at[sl], dbuf.at[:, pl.ds(0, F)])
        pltpu.sync_copy(dbuf, out.at[c].at[sl])


@functools.cache
def _sc_agg_kernel():
    return pl.kernel(
        _sc_agg_body,
        out_type=jax.ShapeDtypeStruct((NC, AGG_ROWS, FD), jnp.float32),
        mesh=plsc.VectorSubcoreMesh(
            core_axis_name="c", subcore_axis_name="s",
            num_cores=NC, num_subcores=NS),
        compiler_params=pltpu.CompilerParams(use_tc_tiling_on_sc=False),
        scratch_types=[
            pltpu.VMEM_SHARED((AGG_ROWS, FP), jnp.float32),
            pltpu.VMEM((G_TILE, B_EDGE), jnp.int32),
            pltpu.VMEM((G_TILE, B_EDGE), jnp.int32),
            pltpu.VMEM((NBUF, B_EDGE, FP), jnp.float32),
            pltpu.VMEM((ZCHUNK, FP), jnp.float32),
            pltpu.VMEM((ZCHUNK, FD), jnp.float32),
            pltpu.SemaphoreType.DMA((NBUF,)),
        ],
    )


def _sc_agg(hw, ei4):
    return _sc_agg_kernel()(hw, ei4)


def _mm_body(x, w, b, hw):
    hw[...] = jnp.dot(x[...], w[...], preferred_element_type=jnp.float32) + b[...]


def _mm_relu_body(x, w, b, res):
    res[...] = jnp.maximum(
        jnp.dot(x[...], w[...], preferred_element_type=jnp.float32) + b[...], 0.0)


def _bn(t, g, be):
    mean = jnp.mean(t, axis=0, keepdims=True)
    var = jnp.mean(t * t, axis=0, keepdims=True) - mean * mean
    return (t - mean) * lax.rsqrt(var + EPS_BN) * g + be


def _k2_body(agg, res, g, be, w, b, hw, hout):
    t = jnp.maximum(agg[0, :N_NODES, :F] + agg[1, :N_NODES, :F], 0.0) + res[...]
    h = _bn(t, g[...], be[...])
    hw[...] = jnp.dot(h, w[...], preferred_element_type=jnp.float32) + b[...]
    hout[...] = h


def _k3_body(agg, res, g, be, wd, bd, out):
    t = jnp.maximum(agg[0, :N_NODES, :F] + agg[1, :N_NODES, :F], 0.0) + res[...]
    h = _bn(t, g[...], be[...])
    logits = jnp.dot(h, wd[...], preferred_element_type=jnp.float32) + bd[...]
    m = jnp.max(logits, axis=1, keepdims=True)
    e = jnp.exp(logits - m)
    out[...] = e / jnp.sum(e, axis=1, keepdims=True)


def _tc_call(body, out_shapes):
    return pl.pallas_call(
        body,
        out_shape=[jax.ShapeDtypeStruct(s, jnp.float32) for s in out_shapes])


def _pad_cols(a, width=FP):
    return jnp.pad(a, ((0, 0), (0, width - a.shape[1])))


def kernel(in_feat, edge_index, W0, b0, Wr0, br0, g0, be0,
           W1, b1, Wr1, br1, g1, be1, Wd, bd):
    ei4 = edge_index.reshape(2, NW, G_TILE, B_EDGE)
    b0p = _pad_cols(b0.reshape(1, F))
    b1p = _pad_cols(b1.reshape(1, F))

    hw0, = _tc_call(_mm_body, [(N_NODES, FP)])(in_feat, _pad_cols(W0), b0p)
    agg0 = _sc_agg(hw0, ei4)
    res0, = _tc_call(_mm_relu_body, [(N_NODES, F)])(
        in_feat, Wr0, br0.reshape(1, F))

    hw1, h1 = _tc_call(_k2_body, [(N_NODES, FP), (N_NODES, F)])(
        agg0, res0, g0.reshape(1, F), be0.reshape(1, F), _pad_cols(W1), b1p)
    agg1 = _sc_agg(hw1, ei4)
    res1, = _tc_call(_mm_relu_body, [(N_NODES, F)])(
        h1, Wr1, br1.reshape(1, F))

    out, = _tc_call(_k3_body, [(N_NODES, 2)])(
        agg1, res1, g1.reshape(1, F), be1.reshape(1, F),
        Wd, bd.reshape(1, 2))
    return out
